# baseline (speedup 1.0000x reference)
import dataclasses
import functools

import jax
import jax.numpy as jnp
from jax import lax
from jax.experimental import pallas as pl
from jax.experimental.pallas import tpu as pltpu
from jax.experimental.pallas import tpu_sc as plsc

N_NODES = 10000
N_EDGES = 320000
FDIM = 128
LANES = 16
NUM_SUBCORES = 16
CHUNK = 128
NUM_CHUNKS = N_EDGES // CHUNK
G = 16
PAD_CHUNKS = -(-NUM_CHUNKS // G) * G
NUM_BLOCKS = PAD_CHUNKS // G
BLOCKS_PER_SUB = -(-NUM_BLOCKS // NUM_SUBCORES)
GUARD = N_NODES
ACC_ROWS = N_NODES + 8
HIST_N = N_NODES + LANES
ROW_TILE = 80
NUM_ROW_TILES = N_NODES // ROW_TILE
ROW_TILES_PER_SUB = -(-NUM_ROW_TILES // NUM_SUBCORES)


def _matmul_bias(feat, w_t, b_row):
    def mm_body(x_ref, w_ref, b_ref, o_ref):
        o_ref[...] = (
            jnp.dot(x_ref[...], w_ref[...], preferred_element_type=jnp.float32)
            + b_ref[...]
        )

    n = feat.shape[0]
    blk = 1000
    return pl.pallas_call(
        mm_body,
        grid=(n // blk,),
        in_specs=[
            pl.BlockSpec((blk, FDIM), lambda i: (i, 0)),
            pl.BlockSpec((FDIM, FDIM), lambda i: (0, 0)),
            pl.BlockSpec((1, FDIM), lambda i: (0, 0)),
        ],
        out_specs=pl.BlockSpec((blk, FDIM), lambda i: (i, 0)),
        out_shape=jax.ShapeDtypeStruct((n, FDIM), jnp.float32),
    )(feat, w_t, b_row)


def _mean_divide(summed, cnt4d):
    blk = 1000

    def div_body(s_ref, c_ref, o_ref):
        cnt = jnp.sum(c_ref[...].reshape(NUM_SUBCORES, blk), axis=0)
        o_ref[...] = s_ref[...] / jnp.maximum(cnt, 1.0)[:, None]

    return pl.pallas_call(
        div_body,
        grid=(N_NODES // blk,),
        in_specs=[
            pl.BlockSpec((blk, FDIM), lambda i: (i, 0)),
            pl.BlockSpec((NUM_SUBCORES, 1, 1, blk), lambda i: (0, i, 0, 0)),
        ],
        out_specs=pl.BlockSpec((blk, FDIM), lambda i: (i, 0)),
        out_shape=jax.ShapeDtypeStruct((N_NODES, FDIM), jnp.float32),
    )(summed, cnt4d)


def _sc_sum_agg(wh_clicks, srcA2, dstA2, wh_cb, srcB2, dstB2):
    mesh = plsc.VectorSubcoreMesh(core_axis_name="c", subcore_axis_name="s")
    f32 = jnp.float32
    cp = pltpu.CompilerParams()
    if "needs_layout_passes" in pltpu.CompilerParams.__dataclass_fields__:
        cp = dataclasses.replace(cp, needs_layout_passes=False)

    @functools.partial(
        pl.kernel,
        mesh=mesh,
        compiler_params=cp,
        out_type=(
            jax.ShapeDtypeStruct((N_NODES, FDIM), f32),
            jax.ShapeDtypeStruct((N_NODES, FDIM), f32),
            jax.ShapeDtypeStruct((NUM_SUBCORES * N_NODES,), f32),
            jax.ShapeDtypeStruct((NUM_SUBCORES * N_NODES,), f32),
        ),
        scratch_types=[
            pltpu.VMEM_SHARED((ACC_ROWS, FDIM), f32),
            pltpu.VMEM((G, CHUNK), jnp.int32),
            pltpu.VMEM((G, CHUNK), jnp.int32),
            pltpu.VMEM((CHUNK, FDIM), f32),
            pltpu.VMEM((CHUNK, FDIM), f32),
            pltpu.VMEM((HIST_N,), f32),
            pltpu.SemaphoreType.DMA,
            pltpu.SemaphoreType.DMA,
        ],
    )
    def agg(whA, sA2, dA2, whB, sB2, dB2,
            sum_user, sum_item, hists_user, hists_item,
            acc, sidx2, didx2, rows0, rows1, hist,
            sem_g, sem_s):
        c = lax.axis_index("c")
        s = lax.axis_index("s")
        bufs = [rows0, rows1]

        @pl.loop(0, CHUNK)
        def _(r):
            for j in range(FDIM // LANES):
                rows0[r, pl.ds(j * LANES, LANES)] = jnp.zeros((LANES,), f32)

        @pl.loop(0, HIST_N // LANES)
        def _(i):
            hist[pl.ds(i * LANES, LANES)] = jnp.zeros((LANES,), f32)

        @pl.loop(0, ROW_TILES_PER_SUB)
        def _(i):
            tid = s + NUM_SUBCORES * i

            @pl.when(tid < NUM_ROW_TILES)
            def _():
                pltpu.sync_copy(rows0.at[pl.ds(0, ROW_TILE)],
                                acc.at[pl.ds(tid * ROW_TILE, ROW_TILE)])

        @pl.when(s == 0)
        def _():
            pltpu.sync_copy(rows0.at[pl.ds(0, 8)],
                            acc.at[pl.ds(N_NODES, 8)])

        plsc.subcore_barrier()

        def hist_update(idx_ref_2d, g):
            for j in range(CHUNK // LANES):
                dvec = idx_ref_2d[g, pl.ds(j * LANES, LANES)]
                plsc.addupdate_scatter(hist, [dvec], jnp.ones((LANES,), f32))

        def accumulate(wh, src2, dst2):
            @pl.loop(0, BLOCKS_PER_SUB)
            def _(i):
                b = s + NUM_SUBCORES * i

                @pl.when(b < NUM_BLOCKS)
                def _():
                    row0 = b * G
                    ih1 = pltpu.async_copy(src2.at[pl.ds(row0, G)], sidx2,
                                           sem_g)
                    ih2 = pltpu.async_copy(dst2.at[pl.ds(row0, G)], didx2,
                                           sem_g)
                    ih1.wait()
                    ih2.wait()
                    gh = [None] * G
                    sh = [None] * G
                    gh[0] = pltpu.async_copy(wh.at[sidx2.at[0]], rows0, sem_g)
                    for g in range(G):
                        gh[g].wait()
                        sh[g] = pltpu.async_copy(
                            bufs[g % 2], acc.at[didx2.at[g]], sem_s, add=True)
                        hist_update(didx2, g)
                        if g < G - 1:
                            if g >= 1:
                                sh[g - 1].wait()
                            gh[g + 1] = pltpu.async_copy(
                                wh.at[sidx2.at[g + 1]], bufs[(g + 1) % 2],
                                sem_g)
                    sh[G - 2].wait()
                    sh[G - 1].wait()

        @pl.when(c == 0)
        def _():
            accumulate(whA, sA2, dA2)

        @pl.when(c == 1)
        def _():
            accumulate(whB, sB2, dB2)

        plsc.subcore_barrier()

        def readout(sum_out, hists_out):
            @pl.loop(0, ROW_TILES_PER_SUB)
            def _(i):
                tid = s + NUM_SUBCORES * i

                @pl.when(tid < NUM_ROW_TILES)
                def _():
                    base = tid * ROW_TILE
                    pltpu.sync_copy(acc.at[pl.ds(base, ROW_TILE)],
                                    rows1.at[pl.ds(0, ROW_TILE)])
                    pltpu.sync_copy(rows1.at[pl.ds(0, ROW_TILE)],
                                    sum_out.at[pl.ds(base, ROW_TILE)])

            pltpu.sync_copy(hist.at[pl.ds(0, N_NODES)],
                            hists_out.at[pl.ds(s * N_NODES, N_NODES)])

        @pl.when(c == 0)
        def _():
            readout(sum_item, hists_item)

        @pl.when(c == 1)
        def _():
            readout(sum_user, hists_user)

    return agg(wh_clicks, srcA2, dstA2, wh_cb, srcB2, dstB2)


def kernel(feat_user, feat_item, edge_index_clicks, edge_index_clicked_by,
           W_clicks, b_clicks, W_clicked_by, b_clicked_by):
    wh_user = _matmul_bias(feat_user, W_clicks.T, b_clicks[None, :])
    wh_item = _matmul_bias(feat_item, W_clicked_by.T, b_clicked_by[None, :])

    src_clicks = edge_index_clicks[0].astype(jnp.int32)
    dst_clicks = edge_index_clicks[1].astype(jnp.int32)
    src_cb = edge_index_clicked_by[0].astype(jnp.int32)
    dst_cb = edge_index_clicked_by[1].astype(jnp.int32)

    npad = PAD_CHUNKS * CHUNK - N_EDGES
    zpad = jnp.zeros((npad,), jnp.int32)
    gpad = jnp.full((npad,), GUARD, jnp.int32)

    def pad2(src_e, dst_e):
        return (jnp.concatenate([src_e, zpad]).reshape(PAD_CHUNKS, CHUNK),
                jnp.concatenate([dst_e, gpad]).reshape(PAD_CHUNKS, CHUNK))

    sA2, dA2 = pad2(src_clicks, dst_clicks)
    sB2, dB2 = pad2(src_cb, dst_cb)
    sum_user, sum_item, hists_user, hists_item = _sc_sum_agg(
        wh_user, sA2, dA2, wh_item, sB2, dB2)

    h_user = _mean_divide(
        sum_user, hists_user.reshape(NUM_SUBCORES, N_NODES // 1000, 1, 1000))
    h_item = _mean_divide(
        sum_item, hists_item.reshape(NUM_SUBCORES, N_NODES // 1000, 1, 1000))
    return (h_user, h_item)


# speedup vs baseline: 1.0843x; 1.0843x over previous
import dataclasses
import functools

import jax
import jax.numpy as jnp
from jax import lax
from jax.experimental import pallas as pl
from jax.experimental.pallas import tpu as pltpu
from jax.experimental.pallas import tpu_sc as plsc

N_NODES = 10000
N_EDGES = 320000
FDIM = 128
LANES = 16
NUM_SUBCORES = 16
CHUNK = 128
NUM_CHUNKS = N_EDGES // CHUNK
G = 8
PAD_CHUNKS = -(-NUM_CHUNKS // G) * G
NUM_BLOCKS = PAD_CHUNKS // G
BLOCKS_PER_SUB = -(-NUM_BLOCKS // NUM_SUBCORES)
GUARD = N_NODES
ACC_ROWS = N_NODES + 8
HIST_N = N_NODES + LANES
ROW_TILE = 40
NUM_ROW_TILES = N_NODES // ROW_TILE
ROW_TILES_PER_SUB = -(-NUM_ROW_TILES // NUM_SUBCORES)


def _matmul_bias(feat, w_t, b_row):
    def mm_body(x_ref, w_ref, b_ref, o_ref):
        o_ref[...] = (
            jnp.dot(x_ref[...], w_ref[...], preferred_element_type=jnp.float32)
            + b_ref[...]
        )

    n = feat.shape[0]
    blk = 1000
    return pl.pallas_call(
        mm_body,
        grid=(n // blk,),
        in_specs=[
            pl.BlockSpec((blk, FDIM), lambda i: (i, 0)),
            pl.BlockSpec((FDIM, FDIM), lambda i: (0, 0)),
            pl.BlockSpec((1, FDIM), lambda i: (0, 0)),
        ],
        out_specs=pl.BlockSpec((blk, FDIM), lambda i: (i, 0)),
        out_shape=jax.ShapeDtypeStruct((n, FDIM), jnp.float32),
    )(feat, w_t, b_row)


def _mean_divide(summed, cnt4d):
    blk = 1000

    def div_body(s_ref, c_ref, o_ref):
        cnt = jnp.sum(c_ref[...].reshape(NUM_SUBCORES, blk), axis=0)
        o_ref[...] = s_ref[...] / jnp.maximum(cnt, 1.0)[:, None]

    return pl.pallas_call(
        div_body,
        grid=(N_NODES // blk,),
        in_specs=[
            pl.BlockSpec((blk, FDIM), lambda i: (i, 0)),
            pl.BlockSpec((NUM_SUBCORES, 1, 1, blk), lambda i: (0, i, 0, 0)),
        ],
        out_specs=pl.BlockSpec((blk, FDIM), lambda i: (i, 0)),
        out_shape=jax.ShapeDtypeStruct((N_NODES, FDIM), jnp.float32),
    )(summed, cnt4d)


def _sc_sum_agg(wh_clicks, srcA2, dstA2, wh_cb, srcB2, dstB2):
    mesh = plsc.VectorSubcoreMesh(core_axis_name="c", subcore_axis_name="s")
    f32 = jnp.float32
    cp = pltpu.CompilerParams()
    if "needs_layout_passes" in pltpu.CompilerParams.__dataclass_fields__:
        cp = dataclasses.replace(cp, needs_layout_passes=False)

    @functools.partial(
        pl.kernel,
        mesh=mesh,
        compiler_params=cp,
        out_type=(
            jax.ShapeDtypeStruct((N_NODES, FDIM), f32),
            jax.ShapeDtypeStruct((N_NODES, FDIM), f32),
            jax.ShapeDtypeStruct((NUM_SUBCORES * N_NODES,), f32),
            jax.ShapeDtypeStruct((NUM_SUBCORES * N_NODES,), f32),
        ),
        scratch_types=[
            pltpu.VMEM_SHARED((ACC_ROWS, FDIM), f32),
            pltpu.VMEM((G, CHUNK), jnp.int32),
            pltpu.VMEM((G, CHUNK), jnp.int32),
            pltpu.VMEM((CHUNK, FDIM), f32),
            pltpu.VMEM((CHUNK, FDIM), f32),
            pltpu.VMEM((HIST_N,), f32),
            pltpu.VMEM((ROW_TILE, FDIM), f32),
            pltpu.SemaphoreType.DMA,
            pltpu.SemaphoreType.DMA,
        ],
    )
    def agg(whA, sA2, dA2, whB, sB2, dB2,
            sum_user, sum_item, hists_user, hists_item,
            acc, sidx2, didx2, rows0, rows1, hist, stage,
            sem_g, sem_s):
        c = lax.axis_index("c")
        s = lax.axis_index("s")
        bufs = [rows0, rows1]

        @pl.loop(0, ROW_TILE)
        def _(r):
            for j in range(FDIM // LANES):
                stage[r, pl.ds(j * LANES, LANES)] = jnp.zeros((LANES,), f32)

        @pl.loop(0, HIST_N // LANES)
        def _(i):
            hist[pl.ds(i * LANES, LANES)] = jnp.zeros((LANES,), f32)

        @pl.loop(0, ROW_TILES_PER_SUB)
        def _(i):
            tid = s + NUM_SUBCORES * i

            @pl.when(tid < NUM_ROW_TILES)
            def _():
                pltpu.sync_copy(stage, acc.at[pl.ds(tid * ROW_TILE, ROW_TILE)])

        @pl.when(s == 0)
        def _():
            pltpu.sync_copy(stage.at[pl.ds(0, 8)],
                            acc.at[pl.ds(N_NODES, 8)])

        plsc.subcore_barrier()

        def hist_update(idx_ref_2d, g):
            for j in range(CHUNK // LANES):
                dvec = idx_ref_2d[g, pl.ds(j * LANES, LANES)]
                plsc.addupdate_scatter(hist, [dvec], jnp.ones((LANES,), f32))

        def accumulate(wh, src2, dst2):
            @pl.loop(0, BLOCKS_PER_SUB)
            def _(i):
                b = s + NUM_SUBCORES * i

                @pl.when(b < NUM_BLOCKS)
                def _():
                    row0 = b * G
                    pltpu.sync_copy(src2.at[pl.ds(row0, G)], sidx2)
                    pltpu.sync_copy(dst2.at[pl.ds(row0, G)], didx2)
                    gh = [None] * G
                    sh = [None] * G
                    gh[0] = pltpu.async_copy(wh.at[sidx2.at[0]], rows0, sem_g)
                    for g in range(G):
                        gh[g].wait()
                        sh[g] = pltpu.async_copy(
                            bufs[g % 2], acc.at[didx2.at[g]], sem_s, add=True)
                        hist_update(didx2, g)
                        if g < G - 1:
                            if g >= 1:
                                sh[g - 1].wait()
                            gh[g + 1] = pltpu.async_copy(
                                wh.at[sidx2.at[g + 1]], bufs[(g + 1) % 2],
                                sem_g)
                    sh[G - 2].wait()
                    sh[G - 1].wait()

        @pl.when(c == 0)
        def _():
            accumulate(whA, sA2, dA2)

        @pl.when(c == 1)
        def _():
            accumulate(whB, sB2, dB2)

        plsc.subcore_barrier()

        def readout(sum_out, hists_out):
            @pl.loop(0, ROW_TILES_PER_SUB)
            def _(i):
                tid = s + NUM_SUBCORES * i

                @pl.when(tid < NUM_ROW_TILES)
                def _():
                    base = tid * ROW_TILE
                    pltpu.sync_copy(acc.at[pl.ds(base, ROW_TILE)],
                                    sum_out.at[pl.ds(base, ROW_TILE)])

            pltpu.sync_copy(hist.at[pl.ds(0, N_NODES)],
                            hists_out.at[pl.ds(s * N_NODES, N_NODES)])

        @pl.when(c == 0)
        def _():
            readout(sum_item, hists_item)

        @pl.when(c == 1)
        def _():
            readout(sum_user, hists_user)

    return agg(wh_clicks, srcA2, dstA2, wh_cb, srcB2, dstB2)


def kernel(feat_user, feat_item, edge_index_clicks, edge_index_clicked_by,
           W_clicks, b_clicks, W_clicked_by, b_clicked_by):
    wh_user = _matmul_bias(feat_user, W_clicks.T, b_clicks[None, :])
    wh_item = _matmul_bias(feat_item, W_clicked_by.T, b_clicked_by[None, :])

    src_clicks = edge_index_clicks[0].astype(jnp.int32)
    dst_clicks = edge_index_clicks[1].astype(jnp.int32)
    src_cb = edge_index_clicked_by[0].astype(jnp.int32)
    dst_cb = edge_index_clicked_by[1].astype(jnp.int32)

    npad = PAD_CHUNKS * CHUNK - N_EDGES
    zpad = jnp.zeros((npad,), jnp.int32)
    gpad = jnp.full((npad,), GUARD, jnp.int32)

    def pad2(src_e, dst_e):
        return (jnp.concatenate([src_e, zpad]).reshape(PAD_CHUNKS, CHUNK),
                jnp.concatenate([dst_e, gpad]).reshape(PAD_CHUNKS, CHUNK))

    sA2, dA2 = pad2(src_clicks, dst_clicks)
    sB2, dB2 = pad2(src_cb, dst_cb)
    sum_user, sum_item, hists_user, hists_item = _sc_sum_agg(
        wh_user, sA2, dA2, wh_item, sB2, dB2)

    h_user = _mean_divide(
        sum_user, hists_user.reshape(NUM_SUBCORES, N_NODES // 1000, 1, 1000))
    h_item = _mean_divide(
        sum_item, hists_item.reshape(NUM_SUBCORES, N_NODES // 1000, 1, 1000))
    return (h_user, h_item)


# speedup vs baseline: 1.1016x; 1.0160x over previous
import dataclasses
import functools

import jax
import jax.numpy as jnp
from jax import lax
from jax.experimental import pallas as pl
from jax.experimental.pallas import tpu as pltpu
from jax.experimental.pallas import tpu_sc as plsc

N_NODES = 10000
N_EDGES = 320000
FDIM = 128
LANES = 16
NUM_SUBCORES = 16
CHUNK = 128
NUM_CHUNKS = N_EDGES // CHUNK
G = 8
PAD_CHUNKS = -(-NUM_CHUNKS // G) * G
NUM_BLOCKS = PAD_CHUNKS // G
BLOCKS_PER_SUB = -(-NUM_BLOCKS // NUM_SUBCORES)
GUARD = N_NODES
ACC_ROWS = N_NODES + 8
HIST_N = N_NODES + LANES
ROW_TILE = 40
NUM_ROW_TILES = N_NODES // ROW_TILE
ROW_TILES_PER_SUB = -(-NUM_ROW_TILES // NUM_SUBCORES)


def _matmul_bias(feat, w_t, b_row):
    def mm_body(x_ref, w_ref, b_ref, o_ref):
        o_ref[...] = (
            jnp.dot(x_ref[...], w_ref[...], preferred_element_type=jnp.float32)
            + b_ref[...]
        )

    n = feat.shape[0]
    blk = 1000
    return pl.pallas_call(
        mm_body,
        grid=(n // blk,),
        in_specs=[
            pl.BlockSpec((blk, FDIM), lambda i: (i, 0)),
            pl.BlockSpec((FDIM, FDIM), lambda i: (0, 0)),
            pl.BlockSpec((1, FDIM), lambda i: (0, 0)),
        ],
        out_specs=pl.BlockSpec((blk, FDIM), lambda i: (i, 0)),
        out_shape=jax.ShapeDtypeStruct((n, FDIM), jnp.float32),
    )(feat, w_t, b_row)


def _mean_divide(summed, cnt4d):
    blk = 1000

    def div_body(s_ref, c_ref, o_ref):
        cnt = jnp.sum(c_ref[...].reshape(NUM_SUBCORES, blk), axis=0)
        o_ref[...] = s_ref[...] / jnp.maximum(cnt, 1.0)[:, None]

    return pl.pallas_call(
        div_body,
        grid=(N_NODES // blk,),
        in_specs=[
            pl.BlockSpec((blk, FDIM), lambda i: (i, 0)),
            pl.BlockSpec((NUM_SUBCORES, 1, 1, blk), lambda i: (0, i, 0, 0)),
        ],
        out_specs=pl.BlockSpec((blk, FDIM), lambda i: (i, 0)),
        out_shape=jax.ShapeDtypeStruct((N_NODES, FDIM), jnp.float32),
    )(summed, cnt4d)


def _sc_sum_agg(wh_clicks, srcA2, dstA2, wh_cb, srcB2, dstB2):
    mesh = plsc.VectorSubcoreMesh(core_axis_name="c", subcore_axis_name="s")
    f32 = jnp.float32
    cp = pltpu.CompilerParams()
    if "needs_layout_passes" in pltpu.CompilerParams.__dataclass_fields__:
        cp = dataclasses.replace(cp, needs_layout_passes=False)

    @functools.partial(
        pl.kernel,
        mesh=mesh,
        compiler_params=cp,
        out_type=(
            jax.ShapeDtypeStruct((N_NODES, FDIM), f32),
            jax.ShapeDtypeStruct((N_NODES, FDIM), f32),
            jax.ShapeDtypeStruct((NUM_SUBCORES * N_NODES,), f32),
            jax.ShapeDtypeStruct((NUM_SUBCORES * N_NODES,), f32),
        ),
        scratch_types=[
            pltpu.VMEM_SHARED((ACC_ROWS, FDIM), f32),
            pltpu.VMEM((G, CHUNK), jnp.int32),
            pltpu.VMEM((G, CHUNK), jnp.int32),
            pltpu.VMEM((CHUNK, FDIM), f32),
            pltpu.VMEM((CHUNK, FDIM), f32),
            pltpu.VMEM((HIST_N,), f32),
            pltpu.VMEM((ROW_TILE, FDIM), f32),
            pltpu.SemaphoreType.DMA,
            pltpu.SemaphoreType.DMA,
        ],
    )
    def agg(whA, sA2, dA2, whB, sB2, dB2,
            sum_user, sum_item, hists_user, hists_item,
            acc, sidx2, didx2, rows0, rows1, hist, stage,
            sem_g, sem_s):
        c = lax.axis_index("c")
        s = lax.axis_index("s")
        bufs = [rows0, rows1]

        @pl.loop(0, ROW_TILE)
        def _(r):
            for j in range(FDIM // LANES):
                stage[r, pl.ds(j * LANES, LANES)] = jnp.zeros((LANES,), f32)

        @pl.loop(0, HIST_N // LANES)
        def _(i):
            hist[pl.ds(i * LANES, LANES)] = jnp.zeros((LANES,), f32)

        @pl.loop(0, ROW_TILES_PER_SUB)
        def _(i):
            tid = s + NUM_SUBCORES * i

            @pl.when(tid < NUM_ROW_TILES)
            def _():
                pltpu.sync_copy(stage, acc.at[pl.ds(tid * ROW_TILE, ROW_TILE)])

        @pl.when(s == 0)
        def _():
            pltpu.sync_copy(stage.at[pl.ds(0, 8)],
                            acc.at[pl.ds(N_NODES, 8)])

        plsc.subcore_barrier()

        def hist_update(idx_ref_2d, g):
            for j in range(CHUNK // LANES):
                dvec = idx_ref_2d[g, pl.ds(j * LANES, LANES)]
                plsc.addupdate_scatter(hist, [dvec], jnp.ones((LANES,), f32))

        def accumulate(wh, src2, dst2):
            @pl.loop(0, BLOCKS_PER_SUB)
            def _(i):
                b = s + NUM_SUBCORES * i

                @pl.when(b < NUM_BLOCKS)
                def _():
                    row0 = b * G
                    pltpu.sync_copy(src2.at[pl.ds(row0, G)], sidx2)
                    pltpu.sync_copy(dst2.at[pl.ds(row0, G)], didx2)
                    gh = [None] * G
                    sh = [None] * G
                    gh[0] = pltpu.async_copy(wh.at[sidx2.at[0]], rows0, sem_g)
                    for g in range(G):
                        gh[g].wait()
                        sh[g] = pltpu.async_copy(
                            bufs[g % 2], acc.at[didx2.at[g]], sem_s, add=True)
                        if g < G - 1:
                            if g >= 1:
                                sh[g - 1].wait()
                            gh[g + 1] = pltpu.async_copy(
                                wh.at[sidx2.at[g + 1]], bufs[(g + 1) % 2],
                                sem_g)
                        hist_update(didx2, g)
                    sh[G - 2].wait()
                    sh[G - 1].wait()

        @pl.when(c == 0)
        def _():
            accumulate(whA, sA2, dA2)

        @pl.when(c == 1)
        def _():
            accumulate(whB, sB2, dB2)

        plsc.subcore_barrier()

        def readout(sum_out, hists_out):
            @pl.loop(0, ROW_TILES_PER_SUB)
            def _(i):
                tid = s + NUM_SUBCORES * i

                @pl.when(tid < NUM_ROW_TILES)
                def _():
                    base = tid * ROW_TILE
                    pltpu.sync_copy(acc.at[pl.ds(base, ROW_TILE)],
                                    sum_out.at[pl.ds(base, ROW_TILE)])

            pltpu.sync_copy(hist.at[pl.ds(0, N_NODES)],
                            hists_out.at[pl.ds(s * N_NODES, N_NODES)])

        @pl.when(c == 0)
        def _():
            readout(sum_item, hists_item)

        @pl.when(c == 1)
        def _():
            readout(sum_user, hists_user)

    return agg(wh_clicks, srcA2, dstA2, wh_cb, srcB2, dstB2)


def kernel(feat_user, feat_item, edge_index_clicks, edge_index_clicked_by,
           W_clicks, b_clicks, W_clicked_by, b_clicked_by):
    wh_user = _matmul_bias(feat_user, W_clicks.T, b_clicks[None, :])
    wh_item = _matmul_bias(feat_item, W_clicked_by.T, b_clicked_by[None, :])

    src_clicks = edge_index_clicks[0].astype(jnp.int32)
    dst_clicks = edge_index_clicks[1].astype(jnp.int32)
    src_cb = edge_index_clicked_by[0].astype(jnp.int32)
    dst_cb = edge_index_clicked_by[1].astype(jnp.int32)

    npad = PAD_CHUNKS * CHUNK - N_EDGES
    zpad = jnp.zeros((npad,), jnp.int32)
    gpad = jnp.full((npad,), GUARD, jnp.int32)

    def pad2(src_e, dst_e):
        return (jnp.concatenate([src_e, zpad]).reshape(PAD_CHUNKS, CHUNK),
                jnp.concatenate([dst_e, gpad]).reshape(PAD_CHUNKS, CHUNK))

    sA2, dA2 = pad2(src_clicks, dst_clicks)
    sB2, dB2 = pad2(src_cb, dst_cb)
    sum_user, sum_item, hists_user, hists_item = _sc_sum_agg(
        wh_user, sA2, dA2, wh_item, sB2, dB2)

    h_user = _mean_divide(
        sum_user, hists_user.reshape(NUM_SUBCORES, N_NODES // 1000, 1, 1000))
    h_item = _mean_divide(
        sum_item, hists_item.reshape(NUM_SUBCORES, N_NODES // 1000, 1, 1000))
    return (h_user, h_item)


# speedup vs baseline: 1.1377x; 1.0328x over previous
import dataclasses
import functools

import jax
import jax.numpy as jnp
from jax import lax
from jax.experimental import pallas as pl
from jax.experimental.pallas import tpu as pltpu
from jax.experimental.pallas import tpu_sc as plsc

N_NODES = 10000
N_EDGES = 320000
FDIM = 128
LANES = 16
NUM_SUBCORES = 16
CHUNK = 128
NUM_CHUNKS = N_EDGES // CHUNK
G = 8
NUM_BLOCKS = NUM_CHUNKS // G
BLOCKS_PER_SUB = -(-NUM_BLOCKS // NUM_SUBCORES)
TAIL_ROW0 = NUM_BLOCKS * G
TAIL_CHUNKS = NUM_CHUNKS - TAIL_ROW0
ACC_ROWS = N_NODES + 8
HIST_N = N_NODES + LANES
ROW_TILE = 40
NUM_ROW_TILES = N_NODES // ROW_TILE
ROW_TILES_PER_SUB = -(-NUM_ROW_TILES // NUM_SUBCORES)


def _matmul_bias(feat, w_t, b_row):
    def mm_body(x_ref, w_ref, b_ref, o_ref):
        o_ref[...] = (
            jnp.dot(x_ref[...], w_ref[...], preferred_element_type=jnp.float32)
            + b_ref[...]
        )

    n = feat.shape[0]
    blk = 1000
    return pl.pallas_call(
        mm_body,
        grid=(n // blk,),
        in_specs=[
            pl.BlockSpec((blk, FDIM), lambda i: (i, 0)),
            pl.BlockSpec((FDIM, FDIM), lambda i: (0, 0)),
            pl.BlockSpec((1, FDIM), lambda i: (0, 0)),
        ],
        out_specs=pl.BlockSpec((blk, FDIM), lambda i: (i, 0)),
        out_shape=jax.ShapeDtypeStruct((n, FDIM), jnp.float32),
    )(feat, w_t, b_row)


def _mean_divide(summed, cnt4d):
    blk = 1000

    def div_body(s_ref, c_ref, o_ref):
        cnt = jnp.sum(c_ref[...].reshape(NUM_SUBCORES, blk), axis=0)
        o_ref[...] = s_ref[...] / jnp.maximum(cnt, 1.0)[:, None]

    return pl.pallas_call(
        div_body,
        grid=(N_NODES // blk,),
        in_specs=[
            pl.BlockSpec((blk, FDIM), lambda i: (i, 0)),
            pl.BlockSpec((NUM_SUBCORES, 1, 1, blk), lambda i: (0, i, 0, 0)),
        ],
        out_specs=pl.BlockSpec((blk, FDIM), lambda i: (i, 0)),
        out_shape=jax.ShapeDtypeStruct((N_NODES, FDIM), jnp.float32),
    )(summed, cnt4d)


def _sc_sum_agg(wh_clicks, srcA2, dstA2, wh_cb, srcB2, dstB2):
    mesh = plsc.VectorSubcoreMesh(core_axis_name="c", subcore_axis_name="s")
    f32 = jnp.float32
    cp = pltpu.CompilerParams()
    if "needs_layout_passes" in pltpu.CompilerParams.__dataclass_fields__:
        cp = dataclasses.replace(cp, needs_layout_passes=False)

    @functools.partial(
        pl.kernel,
        mesh=mesh,
        compiler_params=cp,
        out_type=(
            jax.ShapeDtypeStruct((N_NODES, FDIM), f32),
            jax.ShapeDtypeStruct((N_NODES, FDIM), f32),
            jax.ShapeDtypeStruct((NUM_SUBCORES * N_NODES,), f32),
            jax.ShapeDtypeStruct((NUM_SUBCORES * N_NODES,), f32),
        ),
        scratch_types=[
            pltpu.VMEM_SHARED((ACC_ROWS, FDIM), f32),
            pltpu.VMEM((G, CHUNK), jnp.int32),
            pltpu.VMEM((G, CHUNK), jnp.int32),
            pltpu.VMEM((CHUNK, FDIM), f32),
            pltpu.VMEM((CHUNK, FDIM), f32),
            pltpu.VMEM((HIST_N,), f32),
            pltpu.VMEM((ROW_TILE, FDIM), f32),
            pltpu.SemaphoreType.DMA,
            pltpu.SemaphoreType.DMA,
        ],
    )
    def agg(whA, sA2, dA2, whB, sB2, dB2,
            sum_user, sum_item, hists_user, hists_item,
            acc, sidx2, didx2, rows0, rows1, hist, stage,
            sem_g, sem_s):
        c = lax.axis_index("c")
        s = lax.axis_index("s")
        bufs = [rows0, rows1]

        @pl.loop(0, ROW_TILE)
        def _(r):
            for j in range(FDIM // LANES):
                stage[r, pl.ds(j * LANES, LANES)] = jnp.zeros((LANES,), f32)

        @pl.loop(0, HIST_N // LANES)
        def _(i):
            hist[pl.ds(i * LANES, LANES)] = jnp.zeros((LANES,), f32)

        @pl.loop(0, ROW_TILES_PER_SUB)
        def _(i):
            tid = s + NUM_SUBCORES * i

            @pl.when(tid < NUM_ROW_TILES)
            def _():
                pltpu.sync_copy(stage, acc.at[pl.ds(tid * ROW_TILE, ROW_TILE)])

        @pl.when(s == 0)
        def _():
            pltpu.sync_copy(stage.at[pl.ds(0, 8)],
                            acc.at[pl.ds(N_NODES, 8)])

        plsc.subcore_barrier()

        def hist_update(idx_ref_2d, g):
            for j in range(CHUNK // LANES):
                dvec = idx_ref_2d[g, pl.ds(j * LANES, LANES)]
                plsc.addupdate_scatter(hist, [dvec], jnp.ones((LANES,), f32))

        def accumulate(wh, src2, dst2):
            @pl.loop(0, BLOCKS_PER_SUB)
            def _(i):
                b = s + NUM_SUBCORES * i

                @pl.when(b < NUM_BLOCKS)
                def _():
                    row0 = b * G
                    pltpu.sync_copy(src2.at[pl.ds(row0, G)], sidx2)
                    pltpu.sync_copy(dst2.at[pl.ds(row0, G)], didx2)
                    gh = [None] * G
                    sh = [None] * G
                    gh[0] = pltpu.async_copy(wh.at[sidx2.at[0]], rows0, sem_g)
                    for g in range(G):
                        gh[g].wait()
                        sh[g] = pltpu.async_copy(
                            bufs[g % 2], acc.at[didx2.at[g]], sem_s, add=True)
                        if g < G - 1:
                            if g >= 1:
                                sh[g - 1].wait()
                            gh[g + 1] = pltpu.async_copy(
                                wh.at[sidx2.at[g + 1]], bufs[(g + 1) % 2],
                                sem_g)
                        hist_update(didx2, g)
                    sh[G - 2].wait()
                    sh[G - 1].wait()

            @pl.when(s < TAIL_CHUNKS)
            def _():
                pltpu.sync_copy(src2.at[pl.ds(TAIL_ROW0, TAIL_CHUNKS)],
                                sidx2.at[pl.ds(0, TAIL_CHUNKS)])
                pltpu.sync_copy(dst2.at[pl.ds(TAIL_ROW0, TAIL_CHUNKS)],
                                didx2.at[pl.ds(0, TAIL_CHUNKS)])
                pltpu.async_copy(wh.at[sidx2.at[s]], rows0, sem_g).wait()
                pltpu.sync_copy(rows0, acc.at[didx2.at[s]], add=True)
                hist_update(didx2, s)

        @pl.when(c == 0)
        def _():
            accumulate(whA, sA2, dA2)

        @pl.when(c == 1)
        def _():
            accumulate(whB, sB2, dB2)

        plsc.subcore_barrier()

        def readout(sum_out, hists_out):
            @pl.loop(0, ROW_TILES_PER_SUB)
            def _(i):
                tid = s + NUM_SUBCORES * i

                @pl.when(tid < NUM_ROW_TILES)
                def _():
                    base = tid * ROW_TILE
                    pltpu.sync_copy(acc.at[pl.ds(base, ROW_TILE)],
                                    sum_out.at[pl.ds(base, ROW_TILE)])

            pltpu.sync_copy(hist.at[pl.ds(0, N_NODES)],
                            hists_out.at[pl.ds(s * N_NODES, N_NODES)])

        @pl.when(c == 0)
        def _():
            readout(sum_item, hists_item)

        @pl.when(c == 1)
        def _():
            readout(sum_user, hists_user)

    return agg(wh_clicks, srcA2, dstA2, wh_cb, srcB2, dstB2)


def kernel(feat_user, feat_item, edge_index_clicks, edge_index_clicked_by,
           W_clicks, b_clicks, W_clicked_by, b_clicked_by):
    wh_user = _matmul_bias(feat_user, W_clicks.T, b_clicks[None, :])
    wh_item = _matmul_bias(feat_item, W_clicked_by.T, b_clicked_by[None, :])

    src_clicks = edge_index_clicks[0].astype(jnp.int32)
    dst_clicks = edge_index_clicks[1].astype(jnp.int32)
    src_cb = edge_index_clicked_by[0].astype(jnp.int32)
    dst_cb = edge_index_clicked_by[1].astype(jnp.int32)

    sum_user, sum_item, hists_user, hists_item = _sc_sum_agg(
        wh_user,
        src_clicks.reshape(NUM_CHUNKS, CHUNK),
        dst_clicks.reshape(NUM_CHUNKS, CHUNK),
        wh_item,
        src_cb.reshape(NUM_CHUNKS, CHUNK),
        dst_cb.reshape(NUM_CHUNKS, CHUNK))

    h_user = _mean_divide(
        sum_user, hists_user.reshape(NUM_SUBCORES, N_NODES // 1000, 1, 1000))
    h_item = _mean_divide(
        sum_item, hists_item.reshape(NUM_SUBCORES, N_NODES // 1000, 1, 1000))
    return (h_user, h_item)
